# baseline (speedup 1.0000x reference)
import functools

import jax
import jax.numpy as jnp
from jax import lax
from jax.experimental import pallas as pl
from jax.experimental.pallas import tpu as pltpu
from jax.experimental.pallas import tpu_sc as plsc

N = 10000
E = 160000
NUM_FEAT = 128
F = 256
R = 3
L = 512
C = 16

NC = 2
NS = 16
HALF = F // NC

ECH = 48
BCH = 8
NBLK = 28
NECH = NBLK * BCH
EPT = NECH * ECH
NA = N + NS
RCH = 40
NRCH = N // RCH
RITER = -(-NRCH // NS)

_mesh = plsc.VectorSubcoreMesh(core_axis_name="c", subcore_axis_name="s")


@functools.partial(
    pl.kernel,
    out_type=jax.ShapeDtypeStruct((NC, N, HALF), jnp.float32),
    mesh=_mesh,
    scratch_types=[
        pltpu.VMEM((NRCH, RCH), jnp.int32),
        pltpu.VMEM((RCH, HALF), jnp.float32),
        pltpu.SemaphoreType.DMA,
    ],
)
def _sc_embed(feat_hbm, table_hbm, emb_hbm, idx_v, rows_v, sem):
    c = lax.axis_index("c")
    s = lax.axis_index("s")
    pltpu.sync_copy(feat_hbm, idx_v)
    for t in range(RITER):
        m = s + t * NS

        @pl.when(m < NRCH)
        def _():
            pltpu.async_copy(table_hbm.at[c].at[idx_v.at[m]], rows_v,
                             sem).wait()
            pltpu.sync_copy(rows_v, emb_hbm.at[c].at[pl.ds(m * RCH, RCH)])


@functools.partial(
    pl.kernel,
    out_type=jax.ShapeDtypeStruct((NC, N, HALF), jnp.float32),
    mesh=_mesh,
    scratch_types=[
        pltpu.VMEM((2, BCH, ECH), jnp.int32),
        pltpu.VMEM((2, BCH, ECH), jnp.int32),
        pltpu.VMEM((ECH, HALF), jnp.float32),
        pltpu.VMEM((ECH, HALF), jnp.float32),
        pltpu.VMEM_SHARED((NA, HALF), jnp.float32),
        pltpu.SemaphoreType.DMA,
        pltpu.SemaphoreType.DMA,
        pltpu.SemaphoreType.DMA,
    ],
)
def _sc_segsum(emb_hbm, src_hbm, dst_hbm, agg_hbm,
               sib, dib, buf0, buf1, acc, sem0, sem1, semi):
    c = lax.axis_index("c")
    s = lax.axis_index("s")

    def initbody(t, carry):
        m = s + t * NS

        @pl.when(m < NRCH)
        def _():
            off = pl.multiple_of(m * RCH, 8)
            pltpu.sync_copy(emb_hbm.at[c].at[pl.ds(off, RCH)],
                            acc.at[pl.ds(off, RCH)])

        return carry

    lax.fori_loop(0, RITER, initbody, 0)
    plsc.subcore_barrier()

    pltpu.sync_copy(src_hbm.at[s].at[0], sib.at[0])
    pltpu.sync_copy(dst_hbm.at[s].at[0], dib.at[0])
    pltpu.async_copy(emb_hbm.at[c].at[sib.at[0].at[0]], buf0, sem0)
    bufs = (buf0, buf1)
    sems = (sem0, sem1)

    def blockbody(B, carry):
        b = B % 2
        nb = 1 - b

        @pl.when(B + 1 < NBLK)
        def _():
            pltpu.async_copy(src_hbm.at[s].at[B + 1], sib.at[nb], semi)
            pltpu.async_copy(dst_hbm.at[s].at[B + 1], dib.at[nb], semi)

        for k in range(BCH):
            kb = k % 2
            nkb = 1 - kb
            if k + 1 < BCH:
                pltpu.async_copy(emb_hbm.at[c].at[sib.at[b].at[k + 1]],
                                 bufs[nkb], sems[nkb])
            else:
                @pl.when(B + 1 < NBLK)
                def _():
                    pltpu.make_async_copy(src_hbm.at[s].at[B + 1],
                                          sib.at[nb], semi).wait()
                    pltpu.make_async_copy(dst_hbm.at[s].at[B + 1],
                                          dib.at[nb], semi).wait()
                    pltpu.async_copy(emb_hbm.at[c].at[sib.at[nb].at[0]],
                                     bufs[nkb], sems[nkb])
            pltpu.make_async_copy(emb_hbm.at[c].at[sib.at[b].at[k]],
                                  bufs[kb], sems[kb]).wait()
            pltpu.sync_copy(bufs[kb], acc.at[dib.at[b].at[k]], add=True)
        return carry

    lax.fori_loop(0, NBLK, blockbody, 0)
    plsc.subcore_barrier()

    def outbody(t, carry):
        m = s + t * NS

        @pl.when(m < NRCH)
        def _():
            off = pl.multiple_of(m * RCH, 8)
            pltpu.sync_copy(acc.at[pl.ds(off, RCH)],
                            agg_hbm.at[c].at[pl.ds(off, RCH)])

        return carry

    lax.fori_loop(0, RITER, outbody, 0)


BLK = 1000
_DN = (((1,), (1,)), ((), ()))


def _tc_round_body(agg_ref, wh_ref, bh_ref, wfp_ref, bfp_ref, h_ref, f_ref):
    h = lax.dot_general(agg_ref[0], wh_ref[0], _DN,
                        preferred_element_type=jnp.float32)
    h = h + lax.dot_general(agg_ref[1], wh_ref[1], _DN,
                            preferred_element_type=jnp.float32)
    h = jnp.maximum(h + bh_ref[...], 0.0)
    h_ref[0] = h[:, :HALF]
    h_ref[1] = h[:, HALF:]
    lg = lax.dot_general(h[:, :HALF], wfp_ref[0], _DN,
                         preferred_element_type=jnp.float32)
    lg = lg + lax.dot_general(h[:, HALF:], wfp_ref[1], _DN,
                              preferred_element_type=jnp.float32)
    lg = lg + bfp_ref[...]
    m = jnp.max(lg, axis=-1, keepdims=True)
    e = jnp.exp(lg - m)
    p = e / jnp.sum(e, axis=-1, keepdims=True)

    @pl.when(pl.program_id(0) == 0)
    def _init():
        f_ref[...] = jnp.zeros_like(f_ref)

    f_ref[...] += jnp.sum(p, axis=0, keepdims=True)


def _tc_round(agg3, wh3, bh2, wfp3, bfp2):
    return pl.pallas_call(
        _tc_round_body,
        grid=(N // BLK,),
        in_specs=[
            pl.BlockSpec((NC, BLK, HALF), lambda i: (0, i, 0)),
            pl.BlockSpec((NC, F, HALF), lambda i: (0, 0, 0)),
            pl.BlockSpec((1, F), lambda i: (0, 0)),
            pl.BlockSpec((NC, L, HALF), lambda i: (0, 0, 0)),
            pl.BlockSpec((1, L), lambda i: (0, 0)),
        ],
        out_specs=[
            pl.BlockSpec((NC, BLK, HALF), lambda i: (0, i, 0)),
            pl.BlockSpec((1, L), lambda i: (0, 0)),
        ],
        out_shape=[
            jax.ShapeDtypeStruct((NC, N, HALF), jnp.float32),
            jax.ShapeDtypeStruct((1, L), jnp.float32),
        ],
    )(agg3, wh3, bh2, wfp3, bfp2)


def _tc_final_body(f0_ref, f1_ref, f2_ref, wcl_ref, bcl_ref, out_ref):
    f = f0_ref[...] + f1_ref[...] + f2_ref[...]
    lg = lax.dot_general(f, wcl_ref[...], _DN,
                         preferred_element_type=jnp.float32) + bcl_ref[...]
    m = jnp.max(lg)
    e = jnp.exp(lg - m)
    out_ref[...] = e / jnp.sum(e)


def _tc_final(f0, f1, f2, wcl, bcl2):
    return pl.pallas_call(
        _tc_final_body,
        out_shape=jax.ShapeDtypeStruct((1, C), jnp.float32),
    )(f0, f1, f2, wcl, bcl2)


def kernel(node_feature, edge_index, table, Wh, bh, Wfp, bfp, Wcl, bcl):
    feat = node_feature.astype(jnp.int32).reshape(NRCH, RCH)
    npad = EPT - E // NS
    src = jnp.concatenate(
        [edge_index[0].astype(jnp.int32).reshape(NS, E // NS),
         jnp.zeros((NS, npad), jnp.int32)],
        axis=1).reshape(NS, NBLK, BCH, ECH)
    dst = jnp.concatenate(
        [edge_index[1].astype(jnp.int32).reshape(NS, E // NS),
         jnp.broadcast_to(N + jnp.arange(NS, dtype=jnp.int32)[:, None],
                          (NS, npad))],
        axis=1).reshape(NS, NBLK, BCH, ECH)
    table3 = table.reshape(NUM_FEAT, NC, HALF).transpose(1, 0, 2)
    wh3 = Wh.reshape(R, F, NC, HALF).transpose(0, 2, 1, 3)
    wfp3 = Wfp.reshape(R, L, NC, HALF).transpose(0, 2, 1, 3)

    emb = _sc_embed(feat, table3)
    fparts = []
    for r in range(R):
        agg = _sc_segsum(emb, src, dst)
        emb, fp = _tc_round(agg, wh3[r], bh[r].reshape(1, F),
                            wfp3[r], bfp[r].reshape(1, L))
        fparts.append(fp)
    out = _tc_final(fparts[0], fparts[1], fparts[2], Wcl, bcl.reshape(1, C))
    return out.reshape(C)


# speedup vs baseline: 3.1124x; 3.1124x over previous
import functools

import jax
import jax.numpy as jnp
from jax import lax
from jax.experimental import pallas as pl
from jax.experimental.pallas import tpu as pltpu
from jax.experimental.pallas import tpu_sc as plsc

N = 10000
E = 160000
NUM_FEAT = 128
F = 256
R = 3
L = 512
C = 16

NC = 2
NS = 16
HALF = F // NC
LN = 16

ECH = 80
NECH = 126
EPT = NECH * ECH
NA = NECH * ECH
RITER = 8

_mesh = plsc.VectorSubcoreMesh(core_axis_name="c", subcore_axis_name="s")


@functools.partial(
    pl.kernel,
    out_type=jax.ShapeDtypeStruct((NC, NA, HALF), jnp.float32),
    mesh=_mesh,
    scratch_types=[
        pltpu.VMEM((NECH, ECH), jnp.int32),
        pltpu.VMEM((ECH, HALF), jnp.float32),
        pltpu.SemaphoreType.DMA,
    ],
)
def _sc_embed(feat_hbm, table_hbm, emb_hbm, idx_v, rows_v, sem):
    c = lax.axis_index("c")
    s = lax.axis_index("s")
    pltpu.sync_copy(feat_hbm, idx_v)
    for t in range(RITER):
        m = s + t * NS

        @pl.when(m < NECH)
        def _():
            pltpu.async_copy(table_hbm.at[c].at[idx_v.at[m]], rows_v,
                             sem).wait()
            pltpu.sync_copy(rows_v, emb_hbm.at[c].at[pl.ds(m * ECH, ECH)])


@functools.partial(
    pl.kernel,
    out_type=jax.ShapeDtypeStruct((NC, NA, HALF), jnp.float32),
    mesh=_mesh,
    scratch_types=[
        pltpu.VMEM((NECH, ECH), jnp.int32),
        pltpu.VMEM((2, ECH), jnp.int32),
        pltpu.VMEM((2, ECH), jnp.int32),
        pltpu.VMEM((ECH, HALF), jnp.float32),
        pltpu.VMEM((ECH, HALF), jnp.float32),
        pltpu.VMEM_SHARED((NA, HALF), jnp.float32),
        pltpu.SemaphoreType.DMA,
        pltpu.SemaphoreType.DMA,
    ],
)
def _sc_segsum(emb_hbm, pk_hbm, agg_hbm,
               pk, sidx, didx, buf0, buf1, acc, sem0, sem1):
    c = lax.axis_index("c")
    s = lax.axis_index("s")
    pltpu.sync_copy(pk_hbm.at[s], pk)

    for t in range(RITER):
        m = s + t * NS

        @pl.when(m < NECH)
        def _():
            pltpu.sync_copy(emb_hbm.at[c].at[pl.ds(m * ECH, ECH)], buf0)
            pltpu.sync_copy(buf0, acc.at[pl.ds(m * ECH, ECH)])

    plsc.subcore_barrier()

    bufs = (buf0, buf1)
    sems = (sem0, sem1)

    def unpack(j, bank):
        for k in range(ECH // LN):
            v = pk[j, pl.ds(k * LN, LN)]
            sidx[bank, pl.ds(k * LN, LN)] = jnp.bitwise_and(v, 0xFFFF)
            didx[bank, pl.ds(k * LN, LN)] = lax.shift_right_logical(v, 16)

    def drain(b):
        pltpu.make_async_copy(emb_hbm.at[c].at[pl.ds(0, ECH)], bufs[b],
                              sems[b]).wait()

    unpack(0, 0)
    pltpu.async_copy(emb_hbm.at[c].at[sidx.at[0]], buf0, sem0)

    def body(j2, carry):
        e0 = 2 * j2
        unpack(e0 + 1, 1)
        pltpu.async_copy(emb_hbm.at[c].at[sidx.at[1]], buf1, sem1)
        drain(0)
        pltpu.sync_copy(buf0, acc.at[didx.at[0]], add=True)

        @pl.when(e0 + 2 < NECH)
        def _():
            unpack(e0 + 2, 0)
            pltpu.async_copy(emb_hbm.at[c].at[sidx.at[0]], buf0, sem0)

        drain(1)
        pltpu.sync_copy(buf1, acc.at[didx.at[1]], add=True)
        return carry

    lax.fori_loop(0, NECH // 2, body, 0)
    plsc.subcore_barrier()
    for t in range(RITER):
        m = s + t * NS

        @pl.when(m < NECH)
        def _():
            pltpu.sync_copy(acc.at[pl.ds(m * ECH, ECH)], buf0)
            pltpu.sync_copy(buf0, agg_hbm.at[c].at[pl.ds(m * ECH, ECH)])


BLK = 1000
_DN = (((1,), (1,)), ((), ()))


def _tc_round_body(agg_ref, wh_ref, bh_ref, wfp_ref, bfp_ref, h_ref, f_ref):
    h = lax.dot_general(agg_ref[0], wh_ref[0], _DN,
                        preferred_element_type=jnp.float32)
    h = h + lax.dot_general(agg_ref[1], wh_ref[1], _DN,
                            preferred_element_type=jnp.float32)
    h = jnp.maximum(h + bh_ref[...], 0.0)
    h_ref[0] = h[:, :HALF]
    h_ref[1] = h[:, HALF:]
    lg = lax.dot_general(h[:, :HALF], wfp_ref[0], _DN,
                         preferred_element_type=jnp.float32)
    lg = lg + lax.dot_general(h[:, HALF:], wfp_ref[1], _DN,
                              preferred_element_type=jnp.float32)
    lg = lg + bfp_ref[...]
    m = jnp.max(lg, axis=-1, keepdims=True)
    e = jnp.exp(lg - m)
    p = e / jnp.sum(e, axis=-1, keepdims=True)

    @pl.when(pl.program_id(0) == 0)
    def _init():
        f_ref[...] = jnp.zeros_like(f_ref)

    f_ref[...] += jnp.sum(p, axis=0, keepdims=True)


def _tc_round(agg3, wh3, bh2, wfp3, bfp2):
    return pl.pallas_call(
        _tc_round_body,
        grid=(N // BLK,),
        in_specs=[
            pl.BlockSpec((NC, BLK, HALF), lambda i: (0, i, 0)),
            pl.BlockSpec((NC, F, HALF), lambda i: (0, 0, 0)),
            pl.BlockSpec((1, F), lambda i: (0, 0)),
            pl.BlockSpec((NC, L, HALF), lambda i: (0, 0, 0)),
            pl.BlockSpec((1, L), lambda i: (0, 0)),
        ],
        out_specs=[
            pl.BlockSpec((NC, BLK, HALF), lambda i: (0, i, 0)),
            pl.BlockSpec((1, L), lambda i: (0, 0)),
        ],
        out_shape=[
            jax.ShapeDtypeStruct((NC, NA, HALF), jnp.float32),
            jax.ShapeDtypeStruct((1, L), jnp.float32),
        ],
    )(agg3, wh3, bh2, wfp3, bfp2)


def _tc_final_body(f0_ref, f1_ref, f2_ref, wcl_ref, bcl_ref, out_ref):
    f = f0_ref[...] + f1_ref[...] + f2_ref[...]
    lg = lax.dot_general(f, wcl_ref[...], _DN,
                         preferred_element_type=jnp.float32) + bcl_ref[...]
    m = jnp.max(lg)
    e = jnp.exp(lg - m)
    out_ref[...] = e / jnp.sum(e)


def _tc_final(f0, f1, f2, wcl, bcl2):
    return pl.pallas_call(
        _tc_final_body,
        out_shape=jax.ShapeDtypeStruct((1, C), jnp.float32),
    )(f0, f1, f2, wcl, bcl2)


def kernel(node_feature, edge_index, table, Wh, bh, Wfp, bfp, Wcl, bcl):
    feat = jnp.concatenate(
        [node_feature.astype(jnp.int32),
         jnp.zeros((NA - N,), jnp.int32)]).reshape(NECH, ECH)
    npad = EPT - E // NS
    src = jnp.concatenate(
        [edge_index[0].astype(jnp.int32).reshape(NS, E // NS),
         jnp.zeros((NS, npad), jnp.int32)], axis=1)
    dst = jnp.concatenate(
        [edge_index[1].astype(jnp.int32).reshape(NS, E // NS),
         jnp.broadcast_to(N + jnp.arange(NS, dtype=jnp.int32)[:, None],
                          (NS, npad))], axis=1)
    pk = jnp.bitwise_or(src, jnp.left_shift(dst, 16)).reshape(NS, NECH, ECH)

    table3 = table.reshape(NUM_FEAT, NC, HALF).transpose(1, 0, 2)
    wh3 = Wh.reshape(R, F, NC, HALF).transpose(0, 2, 1, 3)
    wfp3 = Wfp.reshape(R, L, NC, HALF).transpose(0, 2, 1, 3)

    emb = _sc_embed(feat, table3)
    fparts = []
    for r in range(R):
        agg = _sc_segsum(emb, pk)
        emb, fp = _tc_round(agg, wh3[r], bh[r].reshape(1, F),
                            wfp3[r], bfp[r].reshape(1, L))
        fparts.append(fp)
    out = _tc_final(fparts[0], fparts[1], fparts[2], Wcl, bcl.reshape(1, C))
    return out.reshape(C)
